# Initial kernel scaffold; baseline (speedup 1.0000x reference)
import jax
import jax.numpy as jnp
from jax.experimental import pallas as pl


def kernel(x):
    raise NotImplementedError("write your pallas kernel here")


# speedup vs baseline: 1.8908x; 1.8908x over previous
import functools

import jax
import jax.numpy as jnp
from jax import lax
from jax.experimental import pallas as pl
from jax.experimental.pallas import tpu as pltpu
from jax.experimental.pallas import tpu_sc as plsc

B, S, C = 4, 4096, 2048
NUM_CORES = 2
NUM_SUBCORES = 16
NW = NUM_CORES * NUM_SUBCORES
CPW = 128
NCB = C // CPW
NSTRIP = B * NCB
LANES = 16
G = CPW // LANES
S_CHUNK = 256
N_CHUNK = S // S_CHUNK

_mesh = plsc.VectorSubcoreMesh(core_axis_name="c", subcore_axis_name="s")


@functools.partial(
    pl.kernel,
    mesh=_mesh,
    out_type=jax.ShapeDtypeStruct((B, S, C), jnp.float32),
    scratch_types=[pltpu.VMEM((S_CHUNK, CPW), jnp.float32)],
)
def _cumsum_sc(x_hbm, out_hbm, buf):
    wid = lax.axis_index("c") * NUM_SUBCORES + lax.axis_index("s")

    def row_body(s, carries):
        new = []
        for g in range(G):
            v = buf[s, pl.ds(g * LANES, LANES)]
            acc = carries[g] + v
            buf[s, pl.ds(g * LANES, LANES)] = acc
            new.append(acc)
        return tuple(new)

    for strip in range(NSTRIP // NW):
        sid = strip * NW + wid
        b = sid // NCB
        c0 = (sid % NCB) * CPW

        def chunk_body(k, carries):
            s0 = k * S_CHUNK
            pltpu.sync_copy(
                x_hbm.at[b, pl.ds(s0, S_CHUNK), pl.ds(c0, CPW)], buf
            )
            carries = lax.fori_loop(0, S_CHUNK, row_body, carries)
            pltpu.sync_copy(
                buf, out_hbm.at[b, pl.ds(s0, S_CHUNK), pl.ds(c0, CPW)]
            )
            return carries

        zeros = tuple(jnp.zeros((LANES,), jnp.float32) for _ in range(G))
        lax.fori_loop(0, N_CHUNK, chunk_body, zeros)


def kernel(x):
    return _cumsum_sc(x)


# speedup vs baseline: 2.8563x; 1.5107x over previous
import functools

import jax
import jax.numpy as jnp
from jax import lax
from jax.experimental import pallas as pl
from jax.experimental.pallas import tpu as pltpu
from jax.experimental.pallas import tpu_sc as plsc

B, S, C = 4, 4096, 2048
NUM_CORES = 2
NUM_SUBCORES = 16
NW = NUM_CORES * NUM_SUBCORES
CPW = 128
NCB = C // CPW
NSTRIP = B * NCB
LANES = 16
G = CPW // LANES
S_CHUNK = 128
N_CHUNK = S // S_CHUNK
NBUF = 2

_mesh = plsc.VectorSubcoreMesh(core_axis_name="c", subcore_axis_name="s")


@functools.partial(
    pl.kernel,
    mesh=_mesh,
    out_type=jax.ShapeDtypeStruct((B, S, C), jnp.float32),
    scratch_types=(
        [pltpu.VMEM((S_CHUNK, CPW), jnp.float32) for _ in range(2 * NBUF)]
        + [pltpu.SemaphoreType.DMA for _ in range(2 * NBUF)]
    ),
)
def _cumsum_sc(x_hbm, out_hbm, in0, in1, ob0, ob1, is0, is1, os0, os1):
    wid = lax.axis_index("c") * NUM_SUBCORES + lax.axis_index("s")
    in_bufs, out_bufs = [in0, in1], [ob0, ob1]
    in_sems, out_sems = [is0, is1], [os0, os1]

    def row_body(ibuf, obuf):
        def body(s, carries):
            new = []
            for g in range(G):
                acc = carries[g] + ibuf[s, pl.ds(g * LANES, LANES)]
                obuf[s, pl.ds(g * LANES, LANES)] = acc
                new.append(acc)
            return tuple(new)
        return body

    for strip in range(NSTRIP // NW):
        sid = strip * NW + wid
        b = sid // NCB
        c0 = (sid % NCB) * CPW

        def src(k):
            return x_hbm.at[b, pl.ds(k * S_CHUNK, S_CHUNK), pl.ds(c0, CPW)]

        def dst(k):
            return out_hbm.at[b, pl.ds(k * S_CHUNK, S_CHUNK), pl.ds(c0, CPW)]

        for j in range(NBUF):
            pltpu.async_copy(src(j), in_bufs[j], in_sems[j])

        carries = tuple(jnp.zeros((LANES,), jnp.float32) for _ in range(G))

        for j in range(NBUF):
            pltpu.make_async_copy(src(j), in_bufs[j], in_sems[j]).wait()
            carries = lax.fori_loop(
                0, S_CHUNK, row_body(in_bufs[j], out_bufs[j]), carries
            )
            pltpu.async_copy(src(NBUF + j), in_bufs[j], in_sems[j])
            pltpu.async_copy(out_bufs[j], dst(j), out_sems[j])

        def outer_body(g_it, carries):
            k0 = g_it * NBUF
            for j in range(NBUF):
                k = k0 + j
                pltpu.make_async_copy(src(k), in_bufs[j], in_sems[j]).wait()
                pltpu.make_async_copy(out_bufs[j], dst(k), out_sems[j]).wait()
                carries = lax.fori_loop(
                    0, S_CHUNK, row_body(in_bufs[j], out_bufs[j]), carries
                )
                @pl.when(k + NBUF < N_CHUNK)
                def _():
                    pltpu.async_copy(src(k + NBUF), in_bufs[j], in_sems[j])
                pltpu.async_copy(out_bufs[j], dst(k), out_sems[j])
            return carries

        lax.fori_loop(1, N_CHUNK // NBUF, outer_body, carries)

        for j in range(NBUF):
            k = N_CHUNK - NBUF + j
            pltpu.make_async_copy(out_bufs[j], dst(k), out_sems[j]).wait()


def kernel(x):
    return _cumsum_sc(x)
